# Initial kernel scaffold; baseline (speedup 1.0000x reference)
import jax
import jax.numpy as jnp
from jax.experimental import pallas as pl


def kernel(x_, global_attn, ori_indices):
    raise NotImplementedError("write your pallas kernel here")


# speedup vs baseline: 1.0068x; 1.0068x over previous
import functools

import jax
import jax.numpy as jnp
from jax import lax
from jax.experimental import pallas as pl
from jax.experimental.pallas import tpu as pltpu
from jax.experimental.pallas import tpu_sc as plsc

_B, _N, _C = 4, 2048, 1024
_NKEEP = _N // 2
_NP1 = _N + 1
_R = _B * _NP1
_CH = 256
_G = 32
_TILES = 32
_RPT = 256


def _prep_body(ga_row_ref, ga_col_ref, x_ref, src_ref, add2_ref):
    arow = ga_row_ref[0]
    acol = ga_col_ref[0]
    irow = lax.broadcasted_iota(jnp.int32, (1, _N), 1)
    parts = []
    for s in range(_N // _CH):
        a_i = acol[s * _CH:(s + 1) * _CH, :]
        i_i = lax.broadcasted_iota(jnp.int32, (_CH, 1), 0) + s * _CH
        before = (arow > a_i) | ((arow == a_i) & (irow < i_i))
        parts.append(jnp.sum(before.astype(jnp.float32), axis=1, keepdims=True))
    cntcol = jnp.concatenate(parts, axis=0)
    iotacol = lax.broadcasted_iota(jnp.int32, (_N, 1), 0).astype(jnp.float32)
    rankcol = cntcol.astype(jnp.int32)
    src_parts = []
    for c in range(_N // _CH):
        p_i = lax.broadcasted_iota(jnp.int32, (1, _CH), 1) + c * _CH
        onehot = (rankcol == p_i).astype(jnp.float32)
        src_parts.append(jnp.sum(onehot * iotacol, axis=0, keepdims=True))
    src = jnp.concatenate(src_parts, axis=1)
    src_ref[0] = src.astype(jnp.int32)
    wcol = (cntcol >= float(_NKEEP)).astype(jnp.float32) * acol
    x = x_ref[0, 1:, :]
    t = lax.dot_general(wcol, x, (((0,), (0,)), ((), ())),
                        preferred_element_type=jnp.float32,
                        precision=lax.Precision.HIGHEST)
    add2_ref[0] = t * (2.0 / jnp.sum(wcol))


def _make_prep(interpret=False):
    return pl.pallas_call(
        _prep_body,
        grid=(_B,),
        in_specs=[
            pl.BlockSpec((1, 1, _N), lambda b: (b, 0, 0)),
            pl.BlockSpec((1, _N, 1), lambda b: (b, 0, 0)),
            pl.BlockSpec((1, _NP1, _C), lambda b: (b, 0, 0)),
        ],
        out_specs=[
            pl.BlockSpec((1, 1, _N), lambda b: (b, 0, 0)),
            pl.BlockSpec((1, 1, _C), lambda b: (b, 0, 0)),
        ],
        out_shape=[
            jax.ShapeDtypeStruct((_B, 1, _N), jnp.int32),
            jax.ShapeDtypeStruct((_B, 1, _C), jnp.float32),
        ],
        interpret=interpret,
    )


def _add_rows_loop(buf, r, addv):
    def v_body(v, c2):
        sl = pl.ds(v * 16, 16)
        buf[r, sl] = buf[r, sl] + addv[sl]
        return c2
    lax.fori_loop(0, _C // 16, v_body, 0)


def _sc_body(xf, srcf, add2f, out, xbuf, idxbuf, xtail, idxtail,
             addv0, addv1, sem):
    cid = lax.axis_index("c")
    sid = lax.axis_index("s")
    wid = sid * 2 + cid
    base_row = wid * _RPT
    b0 = base_row // _NP1
    bnd = (b0 + 1) * _NP1
    b1 = jnp.minimum(b0 + 1, _B - 1)
    pltpu.sync_copy(add2f.at[pl.ds(b0 * _C, _C)], addv0)
    pltpu.sync_copy(add2f.at[pl.ds(b1 * _C, _C)], addv1)

    for j in range(_RPT // _G):
        r0 = base_row + j * _G
        pltpu.sync_copy(srcf.at[pl.ds(r0, _G)], idxbuf)
        pltpu.async_copy(xf.at[idxbuf], xbuf, sem).wait()

        def g_body(g, carry):
            for rr in range(16):
                r = g * 16 + rr
                grow = r0 + r
                in0 = grow < bnd
                tpos = grow - lax.select(in0, b0 * _NP1, bnd)
                dropped = tpos >= (_NKEEP + 1)

                @pl.when(dropped & in0)
                def _add0(r=r):
                    _add_rows_loop(xbuf, r, addv0)

                @pl.when(dropped & jnp.logical_not(in0))
                def _add1(r=r):
                    _add_rows_loop(xbuf, r, addv1)
            return carry

        lax.fori_loop(0, _G // 16, g_body, 0)
        pltpu.sync_copy(xbuf, out.at[pl.ds(r0, _G)])

    @pl.when(wid == _TILES - 1)
    def _tail():
        t0 = _TILES * _RPT
        pltpu.sync_copy(srcf.at[pl.ds(t0, _R - t0)], idxtail)
        pltpu.async_copy(xf.at[idxtail], xtail, sem).wait()
        for r in range(_R - t0):
            _add_rows_loop(xtail, r, addv0)
        pltpu.sync_copy(xtail, out.at[pl.ds(t0, _R - t0)])


@functools.cache
def _make_sc_gather():
    return functools.partial(
        pl.kernel,
        out_type=jax.ShapeDtypeStruct((_R, _C), jnp.float32),
        mesh=plsc.VectorSubcoreMesh(core_axis_name="c", subcore_axis_name="s"),
        scratch_types=[
            pltpu.VMEM((_G, _C), jnp.float32),
            pltpu.VMEM((_G,), jnp.int32),
            pltpu.VMEM((_R - _TILES * _RPT, _C), jnp.float32),
            pltpu.VMEM((_R - _TILES * _RPT,), jnp.int32),
            pltpu.VMEM((_C,), jnp.float32),
            pltpu.VMEM((_C,), jnp.float32),
            pltpu.SemaphoreType.DMA,
        ],
    )(_sc_body)


def kernel(x_, global_attn, ori_indices):
    del ori_indices
    src, add2 = _make_prep()(
        global_attn.reshape(_B, 1, _N),
        global_attn.reshape(_B, _N, 1),
        x_,
    )
    base = (jnp.arange(_B, dtype=jnp.int32) * _NP1)[:, None]
    src_full = jnp.concatenate(
        [base, src.reshape(_B, _N) + base + 1], axis=1)
    out = _make_sc_gather()(
        x_.reshape(_R, _C),
        src_full.reshape(_R),
        add2.reshape(_B * _C),
    )
    return out.reshape(_B, _NP1, _C)


# speedup vs baseline: 1.9039x; 1.8910x over previous
import functools

import jax
import jax.numpy as jnp
from jax import lax
from jax.experimental import pallas as pl
from jax.experimental.pallas import tpu as pltpu
from jax.experimental.pallas import tpu_sc as plsc

_B, _N, _C = 4, 2048, 1024
_NKEEP = _N // 2
_NP1 = _N + 1
_R = _B * _NP1
_CH = 256
_G = 32
_TILES = 32
_RPT = 256


def _prep_body(ga_row_ref, ga_col_ref, x_ref, src_ref, add2_ref):
    arow = ga_row_ref[0]
    acol = ga_col_ref[0]
    irow = lax.broadcasted_iota(jnp.int32, (1, _N), 1)
    parts = []
    for s in range(_N // _CH):
        a_i = acol[s * _CH:(s + 1) * _CH, :]
        i_i = lax.broadcasted_iota(jnp.int32, (_CH, 1), 0) + s * _CH
        before = (arow > a_i) | ((arow == a_i) & (irow < i_i))
        parts.append(jnp.sum(before.astype(jnp.float32), axis=1, keepdims=True))
    cntcol = jnp.concatenate(parts, axis=0)
    iotacol = lax.broadcasted_iota(jnp.int32, (_N, 1), 0).astype(jnp.float32)
    rankcol = cntcol.astype(jnp.int32)
    src_parts = []
    for c in range(_N // _CH):
        p_i = lax.broadcasted_iota(jnp.int32, (1, _CH), 1) + c * _CH
        onehot = (rankcol == p_i).astype(jnp.float32)
        src_parts.append(jnp.sum(onehot * iotacol, axis=0, keepdims=True))
    src = jnp.concatenate(src_parts, axis=1)
    src_ref[0] = src.astype(jnp.int32)
    wcol = (cntcol >= float(_NKEEP)).astype(jnp.float32) * acol
    x = x_ref[0, 1:, :]
    t = lax.dot_general(wcol, x, (((0,), (0,)), ((), ())),
                        preferred_element_type=jnp.float32,
                        precision=lax.Precision.HIGHEST)
    add2_ref[0] = t * (2.0 / jnp.sum(wcol))


def _make_prep(interpret=False):
    return pl.pallas_call(
        _prep_body,
        grid=(_B,),
        in_specs=[
            pl.BlockSpec((1, 1, _N), lambda b: (b, 0, 0)),
            pl.BlockSpec((1, _N, 1), lambda b: (b, 0, 0)),
            pl.BlockSpec((1, _NP1, _C), lambda b: (b, 0, 0)),
        ],
        out_specs=[
            pl.BlockSpec((1, 1, _N), lambda b: (b, 0, 0)),
            pl.BlockSpec((1, 1, _C), lambda b: (b, 0, 0)),
        ],
        out_shape=[
            jax.ShapeDtypeStruct((_B, 1, _N), jnp.int32),
            jax.ShapeDtypeStruct((_B, 1, _C), jnp.float32),
        ],
        interpret=interpret,
    )


_NCH = _N // _G
_CPW = _NCH // 8


def _sc_body(x3, srcp, tailsrc, add2f, out, xb0, xb1, xb2, idx2d, tidx, addv,
             gs0, gs1, gs2, ss0, ss1, ss2, tsem):
    cid = lax.axis_index("c")
    sid = lax.axis_index("s")
    wid = sid * 2 + cid
    b = wid // 8
    w8 = wid % 8
    xbufs = [xb0, xb1, xb2]
    gsems = [gs0, gs1, gs2]
    ssems = [ss0, ss1, ss2]

    pltpu.sync_copy(add2f.at[pl.ds(b * _C, _C)], addv)
    pltpu.sync_copy(srcp.at[b, w8], idx2d)

    def start_gather(j):
        return pltpu.async_copy(
            x3.at[b].at[idx2d.at[j]], xbufs[j % 3], gsems[j % 3])

    def start_store(j):
        t0 = (8 * j + w8) * _G
        return pltpu.async_copy(
            xbufs[j % 3], out.at[b, pl.ds(t0, _G)], ssems[j % 3])

    g = {}
    s = {}
    g[0] = start_gather(0)
    g[1] = start_gather(1)
    for j in range(_CPW):
        if j + 2 < _CPW:
            if j - 1 >= 0:
                s[j - 1].wait()
            g[j + 2] = start_gather(j + 2)
        g[j].wait()
        xbuf = xbufs[j % 3]
        if j >= 4:
            def v_body(v, c):
                sl = pl.ds(v * 16, 16)
                a = addv[sl]

                def r_body(r, c2):
                    xbuf[r, sl] = xbuf[r, sl] + a
                    return c2
                lax.fori_loop(0, _G, r_body, 0, unroll=4)
                return c
            lax.fori_loop(0, _C // 16, v_body, 0)
            if j == 4:
                @pl.when(w8 == 0)
                def _fix():
                    def v2(v, c):
                        sl = pl.ds(v * 16, 16)
                        xbuf[0, sl] = xbuf[0, sl] - addv[sl]
                        return c
                    lax.fori_loop(0, _C // 16, v2, 0)
        s[j] = start_store(j)
    for j in range(_CPW - 3, _CPW):
        s[j].wait()

    @pl.when(w8 == 7)
    def _tail():
        pltpu.sync_copy(tailsrc.at[b], tidx)
        pltpu.async_copy(x3.at[b].at[tidx.at[0]], xbufs[0], tsem).wait()

        def v_body(v, c):
            sl = pl.ds(v * 16, 16)
            xbufs[0][0, sl] = xbufs[0][0, sl] + addv[sl]
            return c
        lax.fori_loop(0, _C // 16, v_body, 0)
        pltpu.sync_copy(xbufs[0].at[pl.ds(0, 1)], out.at[b, pl.ds(_N, 1)])


@functools.cache
def _make_sc_gather():
    return functools.partial(
        pl.kernel,
        out_type=jax.ShapeDtypeStruct((_B, _NP1, _C), jnp.float32),
        mesh=plsc.VectorSubcoreMesh(core_axis_name="c", subcore_axis_name="s"),
        scratch_types=[
            pltpu.VMEM((_G, _C), jnp.float32),
            pltpu.VMEM((_G, _C), jnp.float32),
            pltpu.VMEM((_G, _C), jnp.float32),
            pltpu.VMEM((8, _G), jnp.int32),
            pltpu.VMEM((1, _G), jnp.int32),
            pltpu.VMEM((_C,), jnp.float32),
            pltpu.SemaphoreType.DMA,
            pltpu.SemaphoreType.DMA,
            pltpu.SemaphoreType.DMA,
            pltpu.SemaphoreType.DMA,
            pltpu.SemaphoreType.DMA,
            pltpu.SemaphoreType.DMA,
            pltpu.SemaphoreType.DMA,
        ],
    )(_sc_body)


def kernel(x_, global_attn, ori_indices):
    del ori_indices
    src, add2 = _make_prep()(
        global_attn.reshape(_B, 1, _N),
        global_attn.reshape(_B, _N, 1),
        x_,
    )
    src_full = jnp.concatenate(
        [jnp.zeros((_B, 1), jnp.int32), src.reshape(_B, _N) + 1],
        axis=1)
    srcp = (src_full[:, :_N].reshape(_B, 8, _CPW, _G)
            .transpose(0, 2, 1, 3))
    tailsrc = jnp.concatenate(
        [src_full[:, _N:], jnp.broadcast_to(
            jnp.arange(1, _G, dtype=jnp.int32)[None], (_B, _G - 1))],
        axis=1).reshape(_B, 1, _G)
    out = _make_sc_gather()(x_, srcp, tailsrc, add2.reshape(_B * _C))
    return out
